# baseline (speedup 1.0000x reference)
import functools

import jax
import jax.numpy as jnp
from jax import lax
from jax.experimental import pallas as pl
from jax.experimental.pallas import tpu as pltpu
from jax.experimental.pallas import tpu_sc as plsc

def _sc_workers():
    info = plsc.get_sparse_core_info()
    return info.num_cores, info.num_subcores

_CHUNK = 128
_NBUF = 8
_LAG = 4

_TBLK = 512
_TGRP = 64


def _transpose_table(wt):
    D, V = wt.shape
    half = _TBLK // 2
    n_groups = (V + _TBLK - 1) // _TBLK
    Vp = n_groups * _TBLK
    grid = (n_groups + _TGRP - 1) // _TGRP

    def body(x_ref, o_ref):
        for t in range(_TGRP):
            x = x_ref[:, t * _TBLK:(t + 1) * _TBLK]
            o_ref[t * half:(t + 1) * half, :] = jnp.concatenate(
                [x[:, :half].T, x[:, half:].T], axis=1
            )

    return pl.pallas_call(
        body,
        grid=(grid,),
        in_specs=[pl.BlockSpec((D, _TGRP * _TBLK), lambda i: (0, i))],
        out_specs=pl.BlockSpec((_TGRP * half, 2 * D), lambda i: (i, 0)),
        out_shape=jax.ShapeDtypeStruct((Vp // 2, 2 * D), jnp.float32),
    )(wt)


def _remap_indices(v):
    k = v & (_TBLK - 1)
    base = v - k
    return (base + jnp.where(k < _TBLK // 2, 2 * k, 2 * k - (_TBLK - 1))).astype(
        jnp.int32
    )


def _relayout_output(flat2, B, L, D):
    NBH = B // 128
    R = NBH * D

    LPB = 2

    def body(x_ref, o_ref):
        eye = jnp.eye(D, dtype=jnp.float32)
        for li in range(LPB):
            x = x_ref[li * R:(li + 1) * R, :]
            t0 = x[:, :D].T
            t1 = x[:, D:].T
            for dh in range(D // 8):
                z0 = t0[dh * 8:(dh + 1) * 8, :].reshape(8, NBH, D)
                z1 = t1[dh * 8:(dh + 1) * 8, :].reshape(8, NBH, D)
                r0 = li * R + dh * 256
                o_ref[r0:r0 + 256, :D] = z0.transpose(1, 0, 2).reshape(8 * NBH, D)
                o_ref[r0:r0 + 256, D:] = z1.transpose(1, 0, 2).reshape(8 * NBH, D)

    return pl.pallas_call(
        body,
        grid=(L // LPB,),
        in_specs=[pl.BlockSpec((LPB * R, 2 * D), lambda l: (l, 0))],
        out_specs=pl.BlockSpec((LPB * R, 2 * D), lambda l: (l, 0)),
        out_shape=jax.ShapeDtypeStruct((L * R, 2 * D), jnp.float32),
    )(flat2)


def kernel(input_, weight):
    B, L = input_.shape
    V, D = weight.shape
    N = B * L
    _NC, _NS = _sc_workers()
    _NW = _NC * _NS
    per_w = N // _NW
    chunks = per_w // _CHUNK
    assert per_w * _NW == N and chunks * _CHUNK == per_w and chunks > _NBUF

    idx_lm = _remap_indices(input_).T.reshape(-1)
    idx3 = idx_lm.reshape(-1, 2, D).transpose(0, 2, 1).reshape(_NW, chunks, _CHUNK)

    w2 = _transpose_table(weight.T)
    Vp = w2.shape[0] * 2
    w_rm = w2.reshape(Vp, D)

    mesh = plsc.VectorSubcoreMesh(core_axis_name="c", subcore_axis_name="s")

    @functools.partial(
        pl.kernel,
        mesh=mesh,
        out_type=jax.ShapeDtypeStruct((N, D), jnp.float32),
        scratch_types=[
            pltpu.VMEM((chunks, _CHUNK), jnp.int32),
            pltpu.VMEM((_NBUF, _CHUNK, D), jnp.float32),
            pltpu.SemaphoreType.DMA,
            pltpu.SemaphoreType.DMA,
        ],
        compiler_params=pltpu.CompilerParams(use_tc_tiling_on_sc=False),
    )
    def gather_kernel(idx_hbm, table_hbm, out_hbm, idx_v, rows_v, gsem, wsem):
        wid = lax.axis_index("s") * _NC + lax.axis_index("c")
        base = wid * per_w
        pltpu.sync_copy(idx_hbm.at[wid], idx_v)

        def start_gather(g):
            pltpu.async_copy(table_hbm.at[idx_v.at[g]], rows_v.at[g % _NBUF], gsem)

        def wb_copy(g):
            return pltpu.make_async_copy(
                rows_v.at[g % _NBUF],
                out_hbm.at[pl.ds(base + g * _CHUNK, _CHUNK)],
                wsem,
            )

        for g in range(_NBUF):
            start_gather(g)
        for w in range(_NBUF - _LAG):
            pltpu.make_async_copy(
                table_hbm.at[idx_v.at[w]], rows_v.at[w % _NBUF], gsem
            ).wait()
            wb_copy(w).start()

        def step(g, carry):
            wb_copy(g - _NBUF).wait()
            start_gather(g)
            w = g - _LAG
            pltpu.make_async_copy(
                table_hbm.at[idx_v.at[w]], rows_v.at[w % _NBUF], gsem
            ).wait()
            wb_copy(w).start()
            return carry

        lax.fori_loop(_NBUF, chunks, step, 0)

        for w in range(chunks - _LAG, chunks):
            pltpu.make_async_copy(
                table_hbm.at[idx_v.at[w]], rows_v.at[w % _NBUF], gsem
            ).wait()
            wb_copy(w).start()
        for w in range(chunks - _NBUF, chunks):
            wb_copy(w).wait()

    out = gather_kernel(idx3, w_rm)
    out2 = _relayout_output(out.reshape(N // 2, 2 * D), B, L, D)
    out5 = out2.reshape(L, D // 8, B // _CHUNK, 8, _CHUNK)
    return out5.transpose(2, 4, 0, 1, 3).reshape(B, L, D)


# speedup vs baseline: 1.2331x; 1.2331x over previous
import functools

import jax
import jax.numpy as jnp
from jax import lax
from jax.experimental import pallas as pl
from jax.experimental.pallas import tpu as pltpu
from jax.experimental.pallas import tpu_sc as plsc

def _sc_workers():
    info = plsc.get_sparse_core_info()
    return info.num_cores, info.num_subcores

_CHUNK = 128
_NBUF = 8
_LAG = 4

_TBLK = 512
_TGRP = 64


def _transpose_table(wt):
    D, V = wt.shape
    half = _TBLK // 2
    n_groups = (V + _TBLK - 1) // _TBLK
    Vp = n_groups * _TBLK
    grid = (n_groups + _TGRP - 1) // _TGRP

    def body(x_ref, o_ref):
        eye = jnp.eye(D, dtype=jnp.float32)
        xt = lax.dot_general(
            x_ref[...], eye, (((0,), (0,)), ((), ())),
            preferred_element_type=jnp.float32,
        )
        for t in range(_TGRP):
            o_ref[t * half:(t + 1) * half, :D] = xt[t * _TBLK:t * _TBLK + half, :]
            o_ref[t * half:(t + 1) * half, D:] = xt[t * _TBLK + half:(t + 1) * _TBLK, :]

    return pl.pallas_call(
        body,
        grid=(grid,),
        in_specs=[pl.BlockSpec((D, _TGRP * _TBLK), lambda i: (0, i))],
        out_specs=pl.BlockSpec((_TGRP * half, 2 * D), lambda i: (i, 0)),
        out_shape=jax.ShapeDtypeStruct((Vp // 2, 2 * D), jnp.float32),
    )(wt)


def _remap_indices(v):
    k = v & (_TBLK - 1)
    base = v - k
    return (base + jnp.where(k < _TBLK // 2, 2 * k, 2 * k - (_TBLK - 1))).astype(
        jnp.int32
    )


def _relayout_output(flat2, B, L, D):
    NBH = B // 128
    R = NBH * D

    LPB = 2

    def body(x_ref, o_ref):
        eye = jnp.eye(D, dtype=jnp.float32)
        for li in range(LPB):
            x = x_ref[li * R:(li + 1) * R, :]
            t0 = lax.dot_general(
                eye, x[:, :D], (((1,), (1,)), ((), ())),
                preferred_element_type=jnp.float32,
            )
            t1 = lax.dot_general(
                eye, x[:, D:], (((1,), (1,)), ((), ())),
                preferred_element_type=jnp.float32,
            )
            for dh in range(D // 8):
                z0 = t0[dh * 8:(dh + 1) * 8, :].reshape(8, NBH, D)
                z1 = t1[dh * 8:(dh + 1) * 8, :].reshape(8, NBH, D)
                r0 = li * R + dh * 256
                o_ref[r0:r0 + 256, :D] = z0.transpose(1, 0, 2).reshape(8 * NBH, D)
                o_ref[r0:r0 + 256, D:] = z1.transpose(1, 0, 2).reshape(8 * NBH, D)

    return pl.pallas_call(
        body,
        grid=(L // LPB,),
        in_specs=[pl.BlockSpec((LPB * R, 2 * D), lambda l: (l, 0))],
        out_specs=pl.BlockSpec((LPB * R, 2 * D), lambda l: (l, 0)),
        out_shape=jax.ShapeDtypeStruct((L * R, 2 * D), jnp.float32),
    )(flat2)


def kernel(input_, weight):
    B, L = input_.shape
    V, D = weight.shape
    N = B * L
    _NC, _NS = _sc_workers()
    _NW = _NC * _NS
    per_w = N // _NW
    chunks = per_w // _CHUNK
    assert per_w * _NW == N and chunks * _CHUNK == per_w and chunks > _NBUF

    idx_lm = _remap_indices(input_).T.reshape(-1)
    idx3 = idx_lm.reshape(-1, 2, D).transpose(0, 2, 1).reshape(_NW, chunks, _CHUNK)

    w2 = _transpose_table(weight.T)
    Vp = w2.shape[0] * 2
    w_rm = w2.reshape(Vp, D)

    mesh = plsc.VectorSubcoreMesh(core_axis_name="c", subcore_axis_name="s")

    @functools.partial(
        pl.kernel,
        mesh=mesh,
        out_type=jax.ShapeDtypeStruct((N, D), jnp.float32),
        scratch_types=[
            pltpu.VMEM((chunks, _CHUNK), jnp.int32),
            pltpu.VMEM((_NBUF, _CHUNK, D), jnp.float32),
            pltpu.SemaphoreType.DMA,
            pltpu.SemaphoreType.DMA,
        ],
        compiler_params=pltpu.CompilerParams(use_tc_tiling_on_sc=False),
    )
    def gather_kernel(idx_hbm, table_hbm, out_hbm, idx_v, rows_v, gsem, wsem):
        wid = lax.axis_index("s") * _NC + lax.axis_index("c")
        base = wid * per_w
        pltpu.sync_copy(idx_hbm.at[wid], idx_v)

        def start_gather(g):
            pltpu.async_copy(table_hbm.at[idx_v.at[g]], rows_v.at[g % _NBUF], gsem)

        def wb_copy(g):
            return pltpu.make_async_copy(
                rows_v.at[g % _NBUF],
                out_hbm.at[pl.ds(base + g * _CHUNK, _CHUNK)],
                wsem,
            )

        for g in range(_NBUF):
            start_gather(g)
        for w in range(_NBUF - _LAG):
            pltpu.make_async_copy(
                table_hbm.at[idx_v.at[w]], rows_v.at[w % _NBUF], gsem
            ).wait()
            wb_copy(w).start()

        def step(g, carry):
            wb_copy(g - _NBUF).wait()
            start_gather(g)
            w = g - _LAG
            pltpu.make_async_copy(
                table_hbm.at[idx_v.at[w]], rows_v.at[w % _NBUF], gsem
            ).wait()
            wb_copy(w).start()
            return carry

        lax.fori_loop(_NBUF, chunks, step, 0)

        for w in range(chunks - _LAG, chunks):
            pltpu.make_async_copy(
                table_hbm.at[idx_v.at[w]], rows_v.at[w % _NBUF], gsem
            ).wait()
            wb_copy(w).start()
        for w in range(chunks - _NBUF, chunks):
            wb_copy(w).wait()

    out = gather_kernel(idx3, w_rm)
    out2 = _relayout_output(out.reshape(N // 2, 2 * D), B, L, D)
    out5 = out2.reshape(L, D // 8, B // _CHUNK, 8, _CHUNK)
    return out5.transpose(2, 4, 0, 1, 3).reshape(B, L, D)


# speedup vs baseline: 1.3164x; 1.0675x over previous
import functools

import jax
import jax.numpy as jnp
from jax import lax
from jax.experimental import pallas as pl
from jax.experimental.pallas import tpu as pltpu
from jax.experimental.pallas import tpu_sc as plsc

def _sc_workers():
    info = plsc.get_sparse_core_info()
    return info.num_cores, info.num_subcores

_CHUNK = 128
_NBUF = 8
_LAG = 4

_TBLK = 512
_TGRP = 64


def _transpose_table(wt):
    D, V = wt.shape
    half = _TBLK // 2
    n_groups = (V + _TBLK - 1) // _TBLK
    Vp = n_groups * _TBLK
    grid = (n_groups + _TGRP - 1) // _TGRP

    def body(x_ref, o_ref):
        eye = jnp.eye(D, dtype=jnp.float32)
        xt = lax.dot_general(
            x_ref[...], eye, (((0,), (0,)), ((), ())),
            preferred_element_type=jnp.float32,
        )
        for t in range(_TGRP):
            o_ref[t * half:(t + 1) * half, :D] = xt[t * _TBLK:t * _TBLK + half, :]
            o_ref[t * half:(t + 1) * half, D:] = xt[t * _TBLK + half:(t + 1) * _TBLK, :]

    return pl.pallas_call(
        body,
        grid=(grid,),
        in_specs=[pl.BlockSpec((D, _TGRP * _TBLK), lambda i: (0, i))],
        out_specs=pl.BlockSpec((_TGRP * half, 2 * D), lambda i: (i, 0)),
        out_shape=jax.ShapeDtypeStruct((Vp // 2, 2 * D), jnp.float32),
    )(wt)


def _remap_indices(v):
    k = v & (_TBLK - 1)
    base = v - k
    return (base + jnp.where(k < _TBLK // 2, 2 * k, 2 * k - (_TBLK - 1))).astype(
        jnp.int32
    )


_LPB = 2


def _relayout_output(flat2, B, L_half, D, L_total, block_off, buf=None):
    NBH = B // 128
    R = NBH * D

    def body(x_ref, *refs):
        o_ref = refs[-1]
        eye = jnp.eye(D, dtype=jnp.float32)
        for li in range(_LPB):
            x = x_ref[li * R:(li + 1) * R, :]
            t0 = lax.dot_general(
                eye, x[:, :D], (((1,), (1,)), ((), ())),
                preferred_element_type=jnp.float32,
            )
            t1 = lax.dot_general(
                eye, x[:, D:], (((1,), (1,)), ((), ())),
                preferred_element_type=jnp.float32,
            )
            for dh in range(D // 8):
                z0 = t0[dh * 8:(dh + 1) * 8, :].reshape(8, NBH, D)
                z1 = t1[dh * 8:(dh + 1) * 8, :].reshape(8, NBH, D)
                r0 = li * R + dh * 256
                o_ref[r0:r0 + 256, :D] = z0.transpose(1, 0, 2).reshape(8 * NBH, D)
                o_ref[r0:r0 + 256, D:] = z1.transpose(1, 0, 2).reshape(8 * NBH, D)

    in_specs = [pl.BlockSpec((_LPB * R, 2 * D), lambda l: (l, 0))]
    args = [flat2]
    aliases = {}
    if buf is not None:
        in_specs.append(pl.BlockSpec(memory_space=pl.ANY))
        args.append(buf)
        aliases = {1: 0}

    return pl.pallas_call(
        body,
        grid=(L_half // _LPB,),
        in_specs=in_specs,
        out_specs=pl.BlockSpec((_LPB * R, 2 * D), lambda l: (l + block_off, 0)),
        out_shape=jax.ShapeDtypeStruct((L_total * R, 2 * D), jnp.float32),
        input_output_aliases=aliases,
    )(*args)


def kernel(input_, weight):
    B, L = input_.shape
    V, D = weight.shape
    N = B * L
    _NC, _NS = _sc_workers()
    _NW = _NC * _NS
    per_w = N // _NW
    chunks = per_w // _CHUNK
    assert per_w * _NW == N and chunks * _CHUNK == per_w and chunks > _NBUF

    idx_lm = _remap_indices(input_).T.reshape(-1)
    idx_tau = idx_lm.reshape(-1, 2, D).transpose(0, 2, 1).reshape(-1)
    halves = 2
    Nh = N // halves
    chunks = Nh // _NW // _CHUNK
    idx_h = [
        idx_tau[h * Nh:(h + 1) * Nh].reshape(_NW, chunks, _CHUNK)
        for h in range(halves)
    ]
    per_w = Nh // _NW

    w2 = _transpose_table(weight.T)
    Vp = w2.shape[0] * 2
    w_rm = w2.reshape(Vp, D)

    mesh = plsc.VectorSubcoreMesh(core_axis_name="c", subcore_axis_name="s")

    @functools.partial(
        pl.kernel,
        mesh=mesh,
        out_type=jax.ShapeDtypeStruct((Nh, D), jnp.float32),
        scratch_types=[
            pltpu.VMEM((chunks, _CHUNK), jnp.int32),
            pltpu.VMEM((_NBUF, _CHUNK, D), jnp.float32),
            pltpu.SemaphoreType.DMA,
            pltpu.SemaphoreType.DMA,
        ],
        compiler_params=pltpu.CompilerParams(use_tc_tiling_on_sc=False),
    )
    def gather_kernel(idx_hbm, table_hbm, out_hbm, idx_v, rows_v, gsem, wsem):
        wid = lax.axis_index("s") * _NC + lax.axis_index("c")
        base = wid * per_w
        pltpu.sync_copy(idx_hbm.at[wid], idx_v)

        def start_gather(g):
            pltpu.async_copy(table_hbm.at[idx_v.at[g]], rows_v.at[g % _NBUF], gsem)

        def wb_copy(g):
            return pltpu.make_async_copy(
                rows_v.at[g % _NBUF],
                out_hbm.at[pl.ds(base + g * _CHUNK, _CHUNK)],
                wsem,
            )

        for g in range(_NBUF):
            start_gather(g)
        for w in range(_NBUF - _LAG):
            pltpu.make_async_copy(
                table_hbm.at[idx_v.at[w]], rows_v.at[w % _NBUF], gsem
            ).wait()
            wb_copy(w).start()

        def step(g, carry):
            wb_copy(g - _NBUF).wait()
            start_gather(g)
            w = g - _LAG
            pltpu.make_async_copy(
                table_hbm.at[idx_v.at[w]], rows_v.at[w % _NBUF], gsem
            ).wait()
            wb_copy(w).start()
            return carry

        lax.fori_loop(_NBUF, chunks, step, 0)

        for w in range(chunks - _LAG, chunks):
            pltpu.make_async_copy(
                table_hbm.at[idx_v.at[w]], rows_v.at[w % _NBUF], gsem
            ).wait()
            wb_copy(w).start()
        for w in range(chunks - _NBUF, chunks):
            wb_copy(w).wait()

    Lh = L // halves
    blocks_per_half = Lh // _LPB
    o1 = gather_kernel(idx_h[0], w_rm)
    o2 = gather_kernel(idx_h[1], w_rm)
    buf = _relayout_output(o1.reshape(Nh // 2, 2 * D), B, Lh, D, L, 0)
    out2 = _relayout_output(
        o2.reshape(Nh // 2, 2 * D), B, Lh, D, L, blocks_per_half, buf=buf
    )
    out5 = out2.reshape(L, D // 8, B // _CHUNK, 8, _CHUNK)
    return out5.transpose(2, 4, 0, 1, 3).reshape(B, L, D)


# speedup vs baseline: 1.3183x; 1.0014x over previous
import functools

import jax
import jax.numpy as jnp
from jax import lax
from jax.experimental import pallas as pl
from jax.experimental.pallas import tpu as pltpu
from jax.experimental.pallas import tpu_sc as plsc

def _sc_workers():
    info = plsc.get_sparse_core_info()
    return info.num_cores, info.num_subcores

_CHUNK = 128
_NBUF = 8
_LAG = 4

_TBLK = 512
_TGRP = 64


def _transpose_table(wt):
    D, V = wt.shape
    half = _TBLK // 2
    n_groups = (V + _TBLK - 1) // _TBLK
    Vp = n_groups * _TBLK
    grid = (n_groups + _TGRP - 1) // _TGRP

    def body(x_ref, o_ref):
        eye = jnp.eye(D, dtype=jnp.float32)
        xt = lax.dot_general(
            x_ref[...], eye, (((0,), (0,)), ((), ())),
            preferred_element_type=jnp.float32,
        )
        for t in range(_TGRP):
            o_ref[t * half:(t + 1) * half, :D] = xt[t * _TBLK:t * _TBLK + half, :]
            o_ref[t * half:(t + 1) * half, D:] = xt[t * _TBLK + half:(t + 1) * _TBLK, :]

    return pl.pallas_call(
        body,
        grid=(grid,),
        in_specs=[pl.BlockSpec((D, _TGRP * _TBLK), lambda i: (0, i))],
        out_specs=pl.BlockSpec((_TGRP * half, 2 * D), lambda i: (i, 0)),
        out_shape=jax.ShapeDtypeStruct((Vp // 2, 2 * D), jnp.float32),
    )(wt)


def _remap_indices(v):
    k = v & (_TBLK - 1)
    base = v - k
    return (base + jnp.where(k < _TBLK // 2, 2 * k, 2 * k - (_TBLK - 1))).astype(
        jnp.int32
    )


_LPB = 2


def _relayout_output(flat2, B, L_half, D, L_total, block_off, buf=None):
    NBH = B // 128
    R = NBH * D

    def body(x_ref, *refs):
        o_ref = refs[-1]
        eye = jnp.eye(D, dtype=jnp.float32)
        for li in range(_LPB):
            x = x_ref[li * R:(li + 1) * R, :]
            t0 = lax.dot_general(
                eye, x[:, :D], (((1,), (1,)), ((), ())),
                preferred_element_type=jnp.float32,
            )
            t1 = lax.dot_general(
                eye, x[:, D:], (((1,), (1,)), ((), ())),
                preferred_element_type=jnp.float32,
            )
            for dh in range(D // 8):
                z0 = t0[dh * 8:(dh + 1) * 8, :].reshape(8, NBH, D)
                z1 = t1[dh * 8:(dh + 1) * 8, :].reshape(8, NBH, D)
                r0 = li * R + dh * 256
                o_ref[r0:r0 + 256, :D] = z0.transpose(1, 0, 2).reshape(8 * NBH, D)
                o_ref[r0:r0 + 256, D:] = z1.transpose(1, 0, 2).reshape(8 * NBH, D)

    in_specs = [pl.BlockSpec((_LPB * R, 2 * D), lambda l: (l, 0))]
    args = [flat2]
    aliases = {}
    if buf is not None:
        in_specs.append(pl.BlockSpec(memory_space=pl.ANY))
        args.append(buf)
        aliases = {1: 0}

    return pl.pallas_call(
        body,
        grid=(L_half // _LPB,),
        in_specs=in_specs,
        out_specs=pl.BlockSpec((_LPB * R, 2 * D), lambda l: (l + block_off, 0)),
        out_shape=jax.ShapeDtypeStruct((L_total * R, 2 * D), jnp.float32),
        input_output_aliases=aliases,
    )(*args)


def kernel(input_, weight):
    B, L = input_.shape
    V, D = weight.shape
    N = B * L
    _NC, _NS = _sc_workers()
    _NW = _NC * _NS
    per_w = N // _NW
    chunks = per_w // _CHUNK
    assert per_w * _NW == N and chunks * _CHUNK == per_w and chunks > _NBUF

    idx_lm = _remap_indices(input_).T.reshape(-1)
    idx_tau = idx_lm.reshape(-1, 2, D).transpose(0, 2, 1).reshape(-1)
    halves = 4
    Nh = N // halves
    chunks = Nh // _NW // _CHUNK
    idx_h = [
        idx_tau[h * Nh:(h + 1) * Nh].reshape(_NW, chunks, _CHUNK)
        for h in range(halves)
    ]
    per_w = Nh // _NW

    w2 = _transpose_table(weight.T)
    Vp = w2.shape[0] * 2
    w_rm = w2.reshape(Vp, D)

    mesh = plsc.VectorSubcoreMesh(core_axis_name="c", subcore_axis_name="s")

    @functools.partial(
        pl.kernel,
        mesh=mesh,
        out_type=jax.ShapeDtypeStruct((Nh, D), jnp.float32),
        scratch_types=[
            pltpu.VMEM((chunks, _CHUNK), jnp.int32),
            pltpu.VMEM((_NBUF, _CHUNK, D), jnp.float32),
            pltpu.SemaphoreType.DMA,
            pltpu.SemaphoreType.DMA,
        ],
        compiler_params=pltpu.CompilerParams(use_tc_tiling_on_sc=False),
    )
    def gather_kernel(idx_hbm, table_hbm, out_hbm, idx_v, rows_v, gsem, wsem):
        wid = lax.axis_index("s") * _NC + lax.axis_index("c")
        base = wid * per_w
        pltpu.sync_copy(idx_hbm.at[wid], idx_v)

        def start_gather(g):
            pltpu.async_copy(table_hbm.at[idx_v.at[g]], rows_v.at[g % _NBUF], gsem)

        def wb_copy(g):
            return pltpu.make_async_copy(
                rows_v.at[g % _NBUF],
                out_hbm.at[pl.ds(base + g * _CHUNK, _CHUNK)],
                wsem,
            )

        for g in range(_NBUF):
            start_gather(g)
        for w in range(_NBUF - _LAG):
            pltpu.make_async_copy(
                table_hbm.at[idx_v.at[w]], rows_v.at[w % _NBUF], gsem
            ).wait()
            wb_copy(w).start()

        def step(g, carry):
            wb_copy(g - _NBUF).wait()
            start_gather(g)
            w = g - _LAG
            pltpu.make_async_copy(
                table_hbm.at[idx_v.at[w]], rows_v.at[w % _NBUF], gsem
            ).wait()
            wb_copy(w).start()
            return carry

        lax.fori_loop(_NBUF, chunks, step, 0)

        for w in range(chunks - _LAG, chunks):
            pltpu.make_async_copy(
                table_hbm.at[idx_v.at[w]], rows_v.at[w % _NBUF], gsem
            ).wait()
            wb_copy(w).start()
        for w in range(chunks - _NBUF, chunks):
            wb_copy(w).wait()

    Lh = L // halves
    blocks_per_half = Lh // _LPB
    outs = [gather_kernel(idx_h[h], w_rm) for h in range(halves)]
    buf = None
    for h in range(halves):
        buf = _relayout_output(
            outs[h].reshape(Nh // 2, 2 * D), B, Lh, D, L,
            h * blocks_per_half, buf=buf,
        )
    out2 = buf
    out5 = out2.reshape(L, D // 8, B // _CHUNK, 8, _CHUNK)
    return out5.transpose(2, 4, 0, 1, 3).reshape(B, L, D)
